# baseline (speedup 1.0000x reference)
import functools

import jax
import jax.numpy as jnp
import numpy as np
from jax import lax
from jax.experimental import pallas as pl
from jax.experimental.pallas import tpu as pltpu
from jax.experimental.pallas import tpu_sc as plsc

_N_FIELDS = 26
_VOCAB = 100000
_EMB = 32
_N_CONT = 13
_B = 16384
_H1 = 256
_H2 = 128

_NW = 32
_R = _B * _N_FIELDS
_RPW = _R // _NW
_NCH = 8
_CH = _RPW // _NCH
_SCW = 128
_SPC = _CH // _SCW

_NSLOT = 28
_ROWS_OUT = _B * _NSLOT
_BPW = _B // _NW
_XRPW = _BPW * 2

_BM = 2048


def _gather_body(table_hbm, idx_hbm, sidx_hbm, xidx_hbm, xc_hbm, out_hbm,
                 idx_v, sidx_v, xidx_v, rows_v, xc_v, sem):
    wid = lax.axis_index("s") * 2 + lax.axis_index("c")
    table2d = table_hbm
    out2d = out_hbm

    pltpu.sync_copy(idx_hbm.at[wid], idx_v)
    pltpu.sync_copy(sidx_hbm.at[wid], sidx_v)
    pltpu.sync_copy(xidx_hbm.at[wid], xidx_v)
    pltpu.sync_copy(xc_hbm.at[pl.ds(wid * _XRPW, _XRPW)], xc_v)

    def chunk(c, carry):
        pltpu.async_copy(table2d.at[idx_v.at[c]], rows_v, sem).wait()

        def scat(j, carry2):
            pltpu.async_copy(
                rows_v.at[pl.ds(j * _SCW, _SCW)],
                out2d.at[sidx_v.at[c * _SPC + j]], sem).wait()
            return carry2

        lax.fori_loop(0, _SPC, scat, 0)
        return carry

    lax.fori_loop(0, _NCH, chunk, 0)

    def xscat(j, carry):
        pltpu.async_copy(
            xc_v.at[pl.ds(j * _SCW, _SCW)],
            out2d.at[xidx_v.at[j]], sem).wait()
        return carry

    lax.fori_loop(0, _XRPW // _SCW, xscat, 0)


_gather = functools.partial(
    pl.kernel,
    out_type=jax.ShapeDtypeStruct((_ROWS_OUT, _EMB), jnp.float32),
    mesh=plsc.VectorSubcoreMesh(core_axis_name="c", subcore_axis_name="s"),
    compiler_params=pltpu.CompilerParams(use_tc_tiling_on_sc=False),
    scratch_types=[
        pltpu.VMEM((_NCH, _CH), jnp.int32),
        pltpu.VMEM((_NCH * _SPC, _SCW), jnp.int32),
        pltpu.VMEM((_XRPW // _SCW, _SCW), jnp.int32),
        pltpu.VMEM((_CH, _EMB), jnp.float32),
        pltpu.VMEM((_XRPW, _EMB), jnp.float32),
        pltpu.SemaphoreType.DMA,
    ],
)(_gather_body)


_SIDX = np.arange(_R, dtype=np.int32)
_SIDX = (_SIDX // _N_FIELDS) * _NSLOT + _SIDX % _N_FIELDS
_SIDX = _SIDX.reshape(_NW, _NCH * _SPC, _SCW)
_XIDX = np.arange(_B * 2, dtype=np.int32)
_XIDX = (_XIDX // 2) * _NSLOT + _N_FIELDS + _XIDX % 2
_XIDX = _XIDX.reshape(_NW, _XRPW // _SCW, _SCW)


def _mlp_body(e_ref, w1_ref, b1_ref, w2_ref, b2_ref, w3_ref, b3_ref, o_ref):
    h1 = jax.nn.sigmoid(
        jnp.dot(e_ref[...], w1_ref[...], preferred_element_type=jnp.float32)
        + b1_ref[...])
    h2 = jax.nn.sigmoid(
        jnp.dot(h1, w2_ref[...], preferred_element_type=jnp.float32)
        + b2_ref[...])
    o_ref[...] = jax.nn.sigmoid(
        jnp.dot(h2, w3_ref[...], preferred_element_type=jnp.float32)
        + b3_ref[...])


def _mlp(e, w1, b1, w2, b2, w3, b3):
    n_in = _NSLOT * _EMB
    grid = _B // _BM
    return pl.pallas_call(
        _mlp_body,
        grid=(grid,),
        in_specs=[
            pl.BlockSpec((_BM, n_in), lambda i: (i, 0)),
            pl.BlockSpec((n_in, _H1), lambda i: (0, 0)),
            pl.BlockSpec((1, _H1), lambda i: (0, 0)),
            pl.BlockSpec((_H1, _H2), lambda i: (0, 0)),
            pl.BlockSpec((1, _H2), lambda i: (0, 0)),
            pl.BlockSpec((_H2, 1), lambda i: (0, 0)),
            pl.BlockSpec((1, 1), lambda i: (0, 0)),
        ],
        out_specs=pl.BlockSpec((_BM, 1), lambda i: (i, 0)),
        out_shape=jax.ShapeDtypeStruct((_B, 1), jnp.float32),
    )(e, w1, b1, w2, b2, w3, b3)


def kernel(x_cat, x_cont, tables, W1, b1, W2, b2, W3, b3):
    offs = (jnp.arange(_N_FIELDS, dtype=jnp.int32) * _VOCAB)[None, :]
    idx = (x_cat.astype(jnp.int32) + offs).reshape(_NW, _NCH, _CH)
    table_flat = tables.reshape(_N_FIELDS * _VOCAB, _EMB)
    xc64 = jnp.pad(x_cont, ((0, 0), (0, 2 * _EMB - _N_CONT))).reshape(
        _B * 2, _EMB)

    e = _gather(table_flat, idx, jnp.asarray(_SIDX), jnp.asarray(_XIDX),
                xc64).reshape(_B, _NSLOT * _EMB)

    w1p = jnp.concatenate(
        [W1, jnp.zeros((_NSLOT * _EMB - W1.shape[0], _H1), jnp.float32)], 0)
    return _mlp(e, w1p, b1[None, :], W2, b2[None, :], W3, b3[None, :])
